# Initial kernel scaffold; baseline (speedup 1.0000x reference)
import jax
import jax.numpy as jnp
from jax.experimental import pallas as pl


def kernel(rel_pos, pos_embed, pos_embed_t):
    raise NotImplementedError("write your pallas kernel here")


# speedup vs baseline: 3.3354x; 3.3354x over previous
import jax
import jax.numpy as jnp
from jax import lax
from jax.experimental import pallas as pl
from jax.experimental.pallas import tpu as pltpu
from jax.experimental.pallas import tpu_sc as plsc

N = 2048 * 2048
NHEAD = 16
MAX_LEN = 1001
GRID = jnp.float32(0.001)
NC, NS, L = 2, 16, 16
NW = NC * NS
PER_W = N // NW
CHUNK = 512
STEPS = PER_W // CHUNK
GROUPS = CHUNK // L
TBL = MAX_LEN * NHEAD


def _body(rel_hbm, t1_hbm, t2_hbm, out_hbm, t1v, t2v, relv, outv):
    wid = lax.axis_index("s") * NC + lax.axis_index("c")
    pltpu.sync_copy(t1_hbm, t1v)
    pltpu.sync_copy(t2_hbm, t2v)
    iota = lax.iota(jnp.int32, L)
    iota3 = iota * 3
    iota16 = iota * 16
    base_el = wid * PER_W

    def step(s, carry):
        el0 = base_el + s * CHUNK
        pltpu.sync_copy(rel_hbm.at[pl.ds(el0 * 3, CHUNK * 3)], relv)

        def group(g, c2):
            e = g * L
            i3 = iota3 + e * 3
            c0 = plsc.load_gather(relv, [i3])
            c1 = plsc.load_gather(relv, [i3 + 1])
            dist = c0 / GRID
            i1 = dist.astype(jnp.int32)
            frac = dist - i1.astype(jnp.float32)
            dist_t = c1 / GRID
            j1 = dist_t.astype(jnp.int32)
            w2t = dist - j1.astype(jnp.float32)
            i1c = jnp.minimum(i1, MAX_LEN - 1) * NHEAD
            i2c = jnp.minimum(i1 + 1, MAX_LEN - 1) * NHEAD
            j1c = jnp.minimum(j1, MAX_LEN - 1) * NHEAD
            j2c = jnp.minimum(j1 + 1, MAX_LEN - 1) * NHEAD
            ob = iota16 + e * 16
            for d in range(NHEAD):
                a1 = plsc.load_gather(t1v, [i1c + d])
                a2 = plsc.load_gather(t1v, [i2c + d])
                b1 = plsc.load_gather(t2v, [j1c + d])
                b2 = plsc.load_gather(t2v, [j2c + d])
                es = a1 + frac * (a2 - a1)
                et = b1 + w2t * (b2 - b1)
                plsc.store_scatter(outv, [ob + d], es * et)
            return c2

        lax.fori_loop(0, GROUPS, group, 0)
        pltpu.sync_copy(outv, out_hbm.at[pl.ds(el0 * NHEAD, CHUNK * NHEAD)])
        return carry

    lax.fori_loop(0, STEPS, step, 0)


def kernel(rel_pos, pos_embed, pos_embed_t):
    rel_flat = rel_pos.reshape(-1)
    t1 = pos_embed.reshape(-1)
    t2 = pos_embed_t.reshape(-1)
    mesh = plsc.VectorSubcoreMesh(core_axis_name="c", subcore_axis_name="s")
    out = pl.kernel(
        _body,
        mesh=mesh,
        compiler_params=pltpu.CompilerParams(needs_layout_passes=False),
        out_type=jax.ShapeDtypeStruct((N * NHEAD,), jnp.float32),
        scratch_types=[
            pltpu.VMEM((TBL,), jnp.float32),
            pltpu.VMEM((TBL,), jnp.float32),
            pltpu.VMEM((CHUNK * 3,), jnp.float32),
            pltpu.VMEM((CHUNK * NHEAD,), jnp.float32),
        ],
    )(rel_flat, t1, t2)
    return out.reshape(1, 2048, 2048, NHEAD)


# speedup vs baseline: 11.0780x; 3.3213x over previous
import jax
import jax.numpy as jnp
import numpy as np
from jax import lax
from jax.experimental import pallas as pl
from jax.experimental.pallas import tpu as pltpu
from jax.experimental.pallas import tpu_sc as plsc

ROWS = 2048
COLS = 2048
N = ROWS * COLS
NHEAD = 16
MAX_LEN = 1001
GRID = np.float32(0.001)
NC, NS, L = 2, 16, 16
NW = NC * NS
PER_W = N // NW
CHUNK = 512
STEPS = PER_W // CHUNK
GROUPS = CHUNK // L
CPR = COLS // CHUNK
TBL = MAX_LEN * NHEAD


def _body(rel0_hbm, rel1_hbm, t1_hbm, t2_hbm, out_hbm, t1v, t2v, relv, outv):
    wid = lax.axis_index("s") * NC + lax.axis_index("c")
    pltpu.sync_copy(t1_hbm, t1v)
    pltpu.sync_copy(t2_hbm, t2v)
    iota = lax.iota(jnp.int32, L)
    iota16 = iota * 16
    zero_v = jnp.zeros((L,), jnp.int32)
    one_v = jnp.ones((L,), jnp.int32)
    base_el = wid * PER_W

    def step(s, carry):
        el0 = base_el + s * CHUNK
        row = el0 // COLS
        col0 = (s % CPR) * CHUNK
        pltpu.sync_copy(rel0_hbm.at[row, pl.ds(col0, CHUNK)], relv.at[0])
        pltpu.sync_copy(rel1_hbm.at[row, pl.ds(col0, CHUNK)], relv.at[1])

        def group(g, c2):
            e = g * L
            e_idx = iota + e
            c0 = plsc.load_gather(relv, [zero_v, e_idx])
            c1 = plsc.load_gather(relv, [one_v, e_idx])
            dist = c0 / GRID
            i1 = dist.astype(jnp.int32)
            frac = dist - i1.astype(jnp.float32)
            dist_t = c1 / GRID
            j1 = dist_t.astype(jnp.int32)
            w2t = dist - j1.astype(jnp.float32)
            i1c = jnp.minimum(i1, MAX_LEN - 1) * NHEAD
            i2c = jnp.minimum(i1 + 1, MAX_LEN - 1) * NHEAD
            j1c = jnp.minimum(j1, MAX_LEN - 1) * NHEAD
            j2c = jnp.minimum(j1 + 1, MAX_LEN - 1) * NHEAD
            ob = iota16 + e * 16
            for d in range(NHEAD):
                a1 = plsc.load_gather(t1v, [i1c + d])
                a2 = plsc.load_gather(t1v, [i2c + d])
                b1 = plsc.load_gather(t2v, [j1c + d])
                b2 = plsc.load_gather(t2v, [j2c + d])
                es = a1 + frac * (a2 - a1)
                et = b1 + w2t * (b2 - b1)
                plsc.store_scatter(outv, [ob + d], es * et)
            return c2

        lax.fori_loop(0, GROUPS, group, 0)
        pltpu.sync_copy(outv, out_hbm.at[pl.ds(el0 * NHEAD, CHUNK * NHEAD)])
        return carry

    lax.fori_loop(0, STEPS, step, 0)


def kernel(rel_pos, pos_embed, pos_embed_t):
    rel0 = rel_pos[0, :, :, 0]
    rel1 = rel_pos[0, :, :, 1]
    t1 = pos_embed.reshape(-1)
    t2 = pos_embed_t.reshape(-1)
    mesh = plsc.VectorSubcoreMesh(core_axis_name="c", subcore_axis_name="s")
    out = pl.kernel(
        _body,
        mesh=mesh,
        compiler_params=pltpu.CompilerParams(needs_layout_passes=False),
        out_type=jax.ShapeDtypeStruct((N * NHEAD,), jnp.float32),
        scratch_types=[
            pltpu.VMEM((TBL,), jnp.float32),
            pltpu.VMEM((TBL,), jnp.float32),
            pltpu.VMEM((2, CHUNK), jnp.float32),
            pltpu.VMEM((CHUNK * NHEAD,), jnp.float32),
        ],
    )(rel0, rel1, t1, t2)
    return out.reshape(1, ROWS, COLS, NHEAD)
